# baseline (speedup 1.0000x reference)
import functools

import jax
import jax.numpy as jnp
from jax import lax
from jax.experimental import pallas as pl
from jax.experimental.pallas import tpu as pltpu
from jax.experimental.pallas import tpu_sc as plsc

DIM = 64
B = 4096
L = 200
VOCAB = 1000000
NW = 32
BPW = B // NW
TD = DIM // 8
TB = B // 128


def _body(x_hbm, lens_hbm, tab_hbm, out_hbm,
          xstage, lens_v, pairidx, rkbuf, maskbuf, rin, tiles, gsem, osem):
    c_ax = lax.axis_index("c")
    s_ax = lax.axis_index("s")
    wid = s_ax * 2 + c_ax

    pltpu.sync_copy(x_hbm.at[pl.ds(wid * (BPW * L), BPW * L)], xstage)
    pltpu.sync_copy(lens_hbm, lens_v)

    lane = lax.iota(jnp.int32, 16)

    def prep(l):
        def prep_body(i, _):
            bl16 = i * 16 + lane
            v16 = plsc.load_gather(xstage, [bl16 * L + l])
            pairidx[pl.ds(i * 16, 16)] = lax.shift_right_logical(v16, 1)
            rkbuf[pl.ds(i * 16, 16)] = (v16 & 1) * 64
            lv16 = plsc.load_gather(lens_v, [wid * BPW + bl16])
            maskbuf[pl.ds(i * 16, 16)] = jnp.where(l < lv16, 1.0, 0.0)
            return 0
        lax.fori_loop(0, BPW // 16, prep_body, 0)

    def fire_gather(l):
        pltpu.async_copy(tab_hbm.at[pairidx], rin, gsem)

    def wait_gather():
        pltpu.make_async_copy(tab_hbm.at[pairidx], rin, gsem).wait()

    def compute(l):
        def comp_body(i, _):
            sb16 = i * 16 + lane
            rk16 = plsc.load_gather(rkbuf, [sb16])
            m16 = plsc.load_gather(maskbuf, [sb16])
            for td in range(TD):
                for sd in range(8):
                    d = td * 8 + sd
                    g16 = plsc.load_gather(rin, [sb16, rk16 + d])
                    tiles[td, sd, pl.ds(i * 16, 16)] = jnp.maximum(g16, 0.0) * m16
            return 0
        lax.fori_loop(0, BPW // 16, comp_body, 0)

    def fire_out(l):
        for td in range(TD):
            pltpu.async_copy(tiles.at[td], out_hbm.at[l, td, wid], osem)

    def wait_out():
        for td in range(TD):
            pltpu.make_async_copy(tiles.at[td], out_hbm.at[0, td, wid], osem).wait()

    def l_body(l, _):
        prep(l)
        fire_gather(l)
        wait_gather()
        compute(l)
        fire_out(l)
        wait_out()
        return 0

    lax.fori_loop(0, L, l_body, 0)


@jax.jit
def _run(xf, x_lens, tpair):
    mesh = plsc.VectorSubcoreMesh(core_axis_name="c", subcore_axis_name="s")
    k = functools.partial(
        pl.kernel,
        mesh=mesh,
        out_type=jax.ShapeDtypeStruct((L, TD, TB, 8, 128), jnp.float32),
        scratch_types=[
            pltpu.VMEM((BPW * L,), jnp.int32),
            pltpu.VMEM((B,), jnp.int32),
            pltpu.VMEM((BPW,), jnp.int32),
            pltpu.VMEM((BPW,), jnp.int32),
            pltpu.VMEM((BPW,), jnp.float32),
            pltpu.VMEM((BPW, 128), jnp.float32),
            pltpu.VMEM((TD, 8, 128), jnp.float32),
            pltpu.SemaphoreType.DMA,
            pltpu.SemaphoreType.DMA,
        ],
        compiler_params=pltpu.CompilerParams(use_tc_tiling_on_sc=True, needs_layout_passes=False),
    )(_body)
    return k(xf, x_lens, tpair)


def kernel(x, x_lens, table):
    xf = x.reshape(B * L)
    tpair = table.reshape(VOCAB // 2, 128)
    out5 = _run(xf, x_lens, tpair)
    out = out5.transpose(2, 4, 0, 1, 3).reshape(B, L, DIM)
    return out


# speedup vs baseline: 1.5081x; 1.5081x over previous
import functools

import jax
import jax.numpy as jnp
from jax import lax
from jax.experimental import pallas as pl
from jax.experimental.pallas import tpu as pltpu
from jax.experimental.pallas import tpu_sc as plsc

DIM = 64
B = 4096
L = 200
VOCAB = 1000000
NW = 32
BPW = B // NW
TD = DIM // 8
TB = B // 128
LGRP = 2
ROWS = LGRP * BPW
NROUND = L // LGRP
NGROUP = NROUND // 4

PACK_CB = 1024


def _pack_body(tt_ref, o_ref):
    t = tt_ref[...].T
    t3 = t.reshape(PACK_CB // 2, 2, 64)
    o_ref[:, 0:64] = t3[:, 0, :]
    o_ref[:, 64:128] = t3[:, 1, :]


@jax.jit
def _pack(tt):
    grid = (VOCAB + PACK_CB - 1) // PACK_CB
    return pl.pallas_call(
        _pack_body,
        grid=(grid,),
        in_specs=[pl.BlockSpec((64, PACK_CB), lambda i: (0, i))],
        out_specs=pl.BlockSpec((PACK_CB // 2, 128), lambda i: (i, 0)),
        out_shape=jax.ShapeDtypeStruct((VOCAB // 2, 128), jnp.float32),
    )(tt)


def _body(xt_hbm, lens_hbm, tab_hbm, out_hbm,
          idxstage, lens_v,
          pi0, pi1, rk0, rk1, mk0, mk1, rin0, rin1, tl0, tl1,
          gsem0, gsem1, osem0, osem1):
    pidx = (pi0, pi1)
    rkb = (rk0, rk1)
    mkb = (mk0, mk1)
    rin = (rin0, rin1)
    tiles = (tl0, tl1)
    gsem = (gsem0, gsem1)
    osem = (osem0, osem1)

    c_ax = lax.axis_index("c")
    s_ax = lax.axis_index("s")
    wid = s_ax * 2 + c_ax

    pltpu.sync_copy(lens_hbm.at[pl.ds(wid * BPW, BPW)], lens_v)
    lane = lax.iota(jnp.int32, 16)

    def stage(q):
        pltpu.sync_copy(
            xt_hbm.at[pl.ds(pl.multiple_of(8 * q, 8), 8),
                      pl.ds(pl.multiple_of(wid * BPW, 128), BPW)],
            idxstage)

    def build(r, b, li_rows):
        l0 = 2 * r
        for li in range(LGRP):
            row = li_rows[li]

            def bb(i, _):
                v16 = idxstage[row, pl.ds(i * 16, 16)]
                pidx[b][pl.ds(li * BPW + i * 16, 16)] = lax.shift_right_logical(v16, 1)
                rkb[b][pl.ds(li * BPW + i * 16, 16)] = (v16 & 1) * 64
                lv16 = lens_v[pl.ds(i * 16, 16)]
                mkb[b][pl.ds(li * BPW + i * 16, 16)] = jnp.where(l0 + li < lv16, 1.0, 0.0)
                return 0

            lax.fori_loop(0, BPW // 16, bb, 0)

    def fire_gather(b):
        pltpu.async_copy(tab_hbm.at[pidx[b].at[pl.ds(0, 128)]],
                         rin[b].at[pl.ds(0, 128)], gsem[b])
        pltpu.async_copy(tab_hbm.at[pidx[b].at[pl.ds(128, 128)]],
                         rin[b].at[pl.ds(128, 128)], gsem[b])

    def wait_gather(b):
        pltpu.make_async_copy(tab_hbm.at[pidx[b].at[pl.ds(0, 128)]],
                              rin[b].at[pl.ds(0, 128)], gsem[b]).wait()
        pltpu.make_async_copy(tab_hbm.at[pidx[b].at[pl.ds(128, 128)]],
                              rin[b].at[pl.ds(128, 128)], gsem[b]).wait()

    def compute(b):
        def cb(it, _):
            i = lax.shift_right_logical(it, 4)
            t = it & 15
            dlo = (lane + t) & 15
            sb16 = i * 16 + lane
            for li in range(LGRP):
                base = li * BPW
                rk16 = plsc.load_gather(rkb[b], [base + sb16])
                m16 = plsc.load_gather(mkb[b], [base + sb16])
                for j in range(DIM // 16):
                    d16 = j * 16 + dlo
                    g16 = plsc.load_gather(rin[b], [base + sb16, rk16 + d16])
                    plsc.store_scatter(tiles[b], [li * 64 + d16, sb16],
                                       jnp.maximum(g16, 0.0) * m16)
            return 0

        lax.fori_loop(0, (BPW // 16) * 16, cb, 0)

    def fire_out(r, b):
        l0 = 2 * r
        for li in range(LGRP):
            for td in range(TD):
                pltpu.async_copy(tiles[b].at[pl.ds(li * 64 + td * 8, 8)],
                                 out_hbm.at[l0 + li, td, wid], osem[b])

    def wait_out(b):
        for li in range(LGRP):
            for td in range(TD):
                pltpu.make_async_copy(tiles[b].at[pl.ds(li * 64 + td * 8, 8)],
                                      out_hbm.at[0, td, wid], osem[b]).wait()

    LI_ROWS = [(2, 3), (4, 5), (6, 7), (0, 1)]

    def round_step(r, k, b, first, fire_next):
        if fire_next:
            build(r + 1, 1 - b, LI_ROWS[k])
            fire_gather(1 - b)
        wait_gather(b)
        if not first:
            wait_out(b)
        compute(b)
        fire_out(r, b)

    def group(q, first=False, last=False):
        for k in range(4):
            r = q * 4 + k
            if k == 3 and not last:
                stage(q + 1)
            round_step(r, k, k % 2, first and k < 2, not (last and k == 3))

    stage(0)
    build(0, 0, (0, 1))
    fire_gather(0)

    group(0, first=True)

    def gbody(q, _):
        for k in range(4):
            r = q * 4 + k
            if k == 3:
                stage(q + 1)
            round_step(r, k, k % 2, False, True)
        return 0

    lax.fori_loop(1, NGROUP - 1, gbody, 0)

    group(NGROUP - 1, last=True)

    wait_out(0)
    wait_out(1)


@jax.jit
def _run(xt, x_lens, tpair):
    mesh = plsc.VectorSubcoreMesh(core_axis_name="c", subcore_axis_name="s")
    k = functools.partial(
        pl.kernel,
        mesh=mesh,
        out_type=jax.ShapeDtypeStruct((L, TD, TB, 8, 128), jnp.float32),
        scratch_types=[
            pltpu.VMEM((8, BPW), jnp.int32),
            pltpu.VMEM((BPW,), jnp.int32),
            pltpu.VMEM((ROWS,), jnp.int32),
            pltpu.VMEM((ROWS,), jnp.int32),
            pltpu.VMEM((ROWS,), jnp.int32),
            pltpu.VMEM((ROWS,), jnp.int32),
            pltpu.VMEM((ROWS,), jnp.float32),
            pltpu.VMEM((ROWS,), jnp.float32),
            pltpu.VMEM((ROWS, 128), jnp.float32),
            pltpu.VMEM((ROWS, 128), jnp.float32),
            pltpu.VMEM((LGRP * 64, 128), jnp.float32),
            pltpu.VMEM((LGRP * 64, 128), jnp.float32),
            pltpu.SemaphoreType.DMA,
            pltpu.SemaphoreType.DMA,
            pltpu.SemaphoreType.DMA,
            pltpu.SemaphoreType.DMA,
        ],
        compiler_params=pltpu.CompilerParams(
            use_tc_tiling_on_sc=True, needs_layout_passes=False
        ),
    )(_body)
    return k(xt, x_lens, tpair)


def kernel(x, x_lens, table):
    xt = x.T
    tpair = _pack(table.T)
    out5 = _run(xt, x_lens, tpair)
    return out5.transpose(2, 4, 0, 1, 3).reshape(B, L, DIM)
